# Initial kernel scaffold; baseline (speedup 1.0000x reference)
import jax
import jax.numpy as jnp
from jax.experimental import pallas as pl


def kernel(x, edge_index, Wl, bl, Wr, br, att, bias, gamma, beta, Wo, bo):
    raise NotImplementedError("write your pallas kernel here")


# speedup vs baseline: 20.0112x; 20.0112x over previous
import dataclasses
import functools

import jax
import jax.numpy as jnp
from jax import lax
from jax.experimental import pallas as pl
from jax.experimental.pallas import tpu as pltpu
from jax.experimental.pallas import tpu_sc as plsc

N = 50000
E = 800000
D = 64
H = 4
Dh = 16
L = 3

HW = D // 2
NSUB = 16
NCORE = 2
WIN = 128
NWIN = E // WIN
WIN_PER_SUB = -(-NWIN // NSUB)
NPAD = 51200
RWIN = 256
NRWIN = NPAD // RWIN
RWIN_PER_SUB = -(-NRWIN // NSUB)
NDEN = NPAD // 8

BLK = 2000
GRID = N // BLK


def _sc_edge_pass(zl, zr, srcs, dsts, attp, zn, zd):
    mesh = plsc.VectorSubcoreMesh(core_axis_name="c", subcore_axis_name="s")
    cp = pltpu.CompilerParams()
    if "needs_layout_passes" in pltpu.CompilerParams.__dataclass_fields__:
        cp = dataclasses.replace(cp, needs_layout_passes=False)
    if "use_tc_tiling_on_sc" in pltpu.CompilerParams.__dataclass_fields__:
        cp = dataclasses.replace(cp, use_tc_tiling_on_sc=False)

    @functools.partial(
        pl.kernel,
        compiler_params=cp,
        out_type=jax.ShapeDtypeStruct((NCORE, NPAD, HW), jnp.float32),
        mesh=mesh,
        scratch_types=[
            pltpu.VMEM_SHARED((NPAD, HW), jnp.float32),
            pltpu.VMEM_SHARED((NDEN, 16), jnp.float32),
            pltpu.VMEM((WIN,), jnp.int32),
            pltpu.VMEM((WIN,), jnp.int32),
            pltpu.VMEM((WIN,), jnp.int32),
            pltpu.VMEM((WIN, HW), jnp.float32),
            pltpu.VMEM((WIN, HW), jnp.float32),
            pltpu.VMEM((WIN, HW), jnp.float32),
            pltpu.VMEM((WIN, 16), jnp.float32),
            pltpu.VMEM((HW, Dh), jnp.float32),
            pltpu.VMEM((RWIN // 4, HW), jnp.float32),
            pltpu.VMEM((RWIN // 8, 16), jnp.float32),
        ],
    )
    def body(zl_r, zr_r, srcs_r, dsts_r, attp_r, zn_r, zd_r, out_r,
             acc, den, sidx, didx, didx8, xi, xj, contrib, dbuf, attv,
             numv, denv):
        c = lax.axis_index("c")
        s = lax.axis_index("s")
        iota = lax.iota(jnp.int32, 16)

        pltpu.sync_copy(attp_r.at[c], attv)

        @pl.loop(0, RWIN_PER_SUB)
        def _z(it):
            rw = s + it * NSUB

            @pl.when(rw < NRWIN)
            def _():
                rb = pl.multiple_of(rw * RWIN, RWIN)
                pltpu.sync_copy(zn_r.at[pl.ds(rb, RWIN)],
                                acc.at[pl.ds(rb, RWIN)])
                db = pl.multiple_of(rw * (RWIN // 8), RWIN // 8)
                pltpu.sync_copy(zd_r.at[pl.ds(db, RWIN // 8)],
                                den.at[pl.ds(db, RWIN // 8)])

        plsc.subcore_barrier()

        @pl.loop(0, WIN_PER_SUB)
        def _win(it):
            w = s + it * NSUB

            @pl.when(w < NWIN)
            def _():
                eb = pl.multiple_of(w * WIN, WIN)
                pltpu.sync_copy(srcs_r.at[pl.ds(eb, WIN)], sidx)
                pltpu.sync_copy(dsts_r.at[pl.ds(eb, WIN)], didx)
                pltpu.sync_copy(zl_r.at[c].at[sidx], xj)
                pltpu.sync_copy(zr_r.at[c].at[didx], xi)

                @pl.loop(0, WIN // 16)
                def _grp(g):
                    rows = g * 16 + iota
                    dv = didx[pl.ds(g * 16, 16)]
                    didx8[pl.ds(g * 16, 16)] = dv >> 3
                    for j in range(16):
                        dbuf[g * 16 + j, :] = jnp.zeros((16,), jnp.float32)
                    dlo = (dv & 7) * 2
                    for h in range(2):
                        alpha = jnp.zeros((16,), jnp.float32)
                        for k in range(16):
                            col = jnp.full((16,), h * Dh + k, jnp.int32)
                            vi = plsc.load_gather(xi, [rows, col])
                            vj = plsc.load_gather(xj, [rows, col])
                            v = vi + vj
                            t = jnp.maximum(v, 0.0) + 0.2 * jnp.minimum(v, 0.0)
                            alpha = alpha + attv[h * Dh + k] * t
                        ex = jnp.exp(alpha)
                        plsc.store_scatter(dbuf, [rows, dlo + h], ex)
                        for k in range(16):
                            col = jnp.full((16,), h * Dh + k, jnp.int32)
                            vj = plsc.load_gather(xj, [rows, col])
                            plsc.store_scatter(contrib, [rows, col], vj * ex)

                pltpu.sync_copy(contrib, acc.at[didx], add=True)
                pltpu.sync_copy(dbuf, den.at[didx8], add=True)

        plsc.subcore_barrier()

        @pl.loop(0, RWIN_PER_SUB)
        def _rd(it):
            rw = s + it * NSUB

            @pl.when(rw < NRWIN)
            def _():
                rb = pl.multiple_of(rw * RWIN, RWIN)
                db = pl.multiple_of(rw * (RWIN // 8), RWIN // 8)
                pltpu.sync_copy(den.at[pl.ds(db, RWIN // 8)], denv)
                for q in range(4):
                    qb = pl.multiple_of(rb + q * (RWIN // 4), RWIN // 4)
                    pltpu.sync_copy(acc.at[pl.ds(qb, RWIN // 4)], numv)

                    @pl.loop(0, RWIN // 4)
                    def _row(r):
                        rg = q * (RWIN // 4) + r
                        rsp = jnp.full((16,), 0, jnp.int32) + (rg >> 3)
                        for h in range(2):
                            csp = jnp.full((16,), 0, jnp.int32) + \
                                ((rg & 7) * 2 + h)
                            dsp = plsc.load_gather(denv, [rsp, csp])
                            sl = (r, pl.ds(h * Dh, Dh))
                            numv[sl] = numv[sl] / (dsp + 1e-16)

                    pltpu.sync_copy(numv, out_r.at[c].at[pl.ds(qb, RWIN // 4)])

    return body(zl, zr, srcs, dsts, attp, zn, zd)


def _split_body(z, ref):
    ref[0] = z[:, :HW]
    ref[1] = z[:, HW:]


_W_SPEC = pl.BlockSpec((D, D), lambda i: (0, 0))
_V_SPEC = pl.BlockSpec((1, D), lambda i: (0, 0))
_H_SPEC = pl.BlockSpec((BLK, D), lambda i: (i, 0))
_Z_SPEC = pl.BlockSpec((2, BLK, HW), lambda i: (0, i, 0))
_Z_SHAPE = jax.ShapeDtypeStruct((2, N, HW), jnp.float32)


def _tc_pre(x, Wl0, bl0, Wr0, br0):
    def body(x_ref, wl_ref, bl_ref, wr_ref, br_ref, zl_ref, zr_ref):
        xb = x_ref[...]
        zl = jnp.dot(xb, wl_ref[...], preferred_element_type=jnp.float32) + bl_ref[...]
        zr = jnp.dot(xb, wr_ref[...], preferred_element_type=jnp.float32) + br_ref[...]
        _split_body(zl, zl_ref)
        _split_body(zr, zr_ref)

    return pl.pallas_call(
        body, grid=(GRID,),
        in_specs=[_H_SPEC, _W_SPEC, _V_SPEC, _W_SPEC, _V_SPEC],
        out_specs=[_Z_SPEC, _Z_SPEC],
        out_shape=[_Z_SHAPE, _Z_SHAPE],
    )(x, Wl0, bl0.reshape(1, D), Wr0, br0.reshape(1, D))


def _norm_block(h_ref, sc_ref, bias_ref, gamma_ref, beta_ref):
    agg = jnp.concatenate([sc_ref[0], sc_ref[1]], axis=-1)
    out = agg + bias_ref[...] + h_ref[...]
    mu = jnp.mean(out, axis=-1, keepdims=True)
    var = jnp.mean((out - mu) ** 2, axis=-1, keepdims=True)
    return (out - mu) * lax.rsqrt(var + 1e-5) * gamma_ref[...] + beta_ref[...]


def _tc_mid(h, scout, bias_i, gamma_i, beta_i, Wln, bln, Wrn, brn):
    def body(h_ref, sc_ref, bias_ref, gamma_ref, beta_ref,
             wl_ref, bl_ref, wr_ref, br_ref, hn_ref, zl_ref, zr_ref):
        hn = _norm_block(h_ref, sc_ref, bias_ref, gamma_ref, beta_ref)
        hn_ref[...] = hn
        zl = jnp.dot(hn, wl_ref[...], preferred_element_type=jnp.float32) + bl_ref[...]
        zr = jnp.dot(hn, wr_ref[...], preferred_element_type=jnp.float32) + br_ref[...]
        _split_body(zl, zl_ref)
        _split_body(zr, zr_ref)

    return pl.pallas_call(
        body, grid=(GRID,),
        in_specs=[_H_SPEC, _Z_SPEC, _V_SPEC, _V_SPEC, _V_SPEC,
                  _W_SPEC, _V_SPEC, _W_SPEC, _V_SPEC],
        out_specs=[_H_SPEC, _Z_SPEC, _Z_SPEC],
        out_shape=[jax.ShapeDtypeStruct((N, D), jnp.float32), _Z_SHAPE, _Z_SHAPE],
    )(h, scout, bias_i.reshape(1, D), gamma_i.reshape(1, D), beta_i.reshape(1, D),
      Wln, bln.reshape(1, D), Wrn, brn.reshape(1, D))


def _tc_post(h, scout, bias_i, gamma_i, beta_i, Wo, bo):
    def body(h_ref, sc_ref, bias_ref, gamma_ref, beta_ref,
             wo_ref, bo_ref, out_ref, acc_ref):
        hn = _norm_block(h_ref, sc_ref, bias_ref, gamma_ref, beta_ref)
        psum = jnp.sum(hn, axis=0, keepdims=True)
        i = pl.program_id(0)

        @pl.when(i == 0)
        def _():
            acc_ref[...] = psum

        @pl.when(i > 0)
        def _():
            acc_ref[...] += psum

        @pl.when(i == GRID - 1)
        def _():
            pooled = acc_ref[...] * (1.0 / N)
            out_ref[...] = (jnp.dot(pooled, wo_ref[...],
                                    preferred_element_type=jnp.float32)
                            + bo_ref[...])

    return pl.pallas_call(
        body, grid=(GRID,),
        in_specs=[_H_SPEC, _Z_SPEC, _V_SPEC, _V_SPEC, _V_SPEC, _W_SPEC, _V_SPEC],
        out_specs=pl.BlockSpec((1, D), lambda i: (0, 0)),
        out_shape=jax.ShapeDtypeStruct((1, D), jnp.float32),
        scratch_shapes=[pltpu.VMEM((1, D), jnp.float32)],
    )(h, scout, bias_i.reshape(1, D), gamma_i.reshape(1, D),
      beta_i.reshape(1, D), Wo, bo.reshape(1, D))


def kernel(x, edge_index, Wl, bl, Wr, br, att, bias, gamma, beta, Wo, bo):
    srcs = edge_index[0]
    dsts = edge_index[1]
    zn = jnp.zeros((NPAD, HW), jnp.float32)
    zd = jnp.zeros((NDEN, 16), jnp.float32)

    h = x
    zl_t, zr_t = _tc_pre(x, Wl[0], bl[0], Wr[0], br[0])
    out = None
    for i in range(L):
        a2 = att[i].reshape(NCORE, 2, Dh)
        attp = jnp.broadcast_to(a2[..., None], (NCORE, 2, Dh, 16))
        attp = attp.reshape(NCORE, HW, Dh)
        scout = _sc_edge_pass(zl_t, zr_t, srcs, dsts, attp, zn, zd)
        if i < L - 1:
            h, zl_t, zr_t = _tc_mid(h, scout, bias[i], gamma[i], beta[i],
                                    Wl[i + 1], bl[i + 1], Wr[i + 1], br[i + 1])
        else:
            out = _tc_post(h, scout, bias[i], gamma[i], beta[i], Wo, bo)
    return out


# speedup vs baseline: 22.1603x; 1.1074x over previous
import dataclasses
import functools

import jax
import jax.numpy as jnp
from jax import lax
from jax.experimental import pallas as pl
from jax.experimental.pallas import tpu as pltpu
from jax.experimental.pallas import tpu_sc as plsc

N = 50000
E = 800000
D = 64
H = 4
Dh = 16
L = 3

HW = D // 2
NSUB = 16
NCORE = 2
WIN = 64
NPAD = 51200
EPAD = 800768
NWINP = EPAD // WIN
WPS = NWINP // NSUB
PIPE_ITERS = WPS + 2
RWIN = 256
NRWIN = NPAD // RWIN
RWIN_PER_SUB = -(-NRWIN // NSUB)
NDEN = NPAD // 8

BLK = 2000
GRID = N // BLK


def _sc_edge_pass(zl, zr, srcs, dsts, attp, zn, zd):
    mesh = plsc.VectorSubcoreMesh(core_axis_name="c", subcore_axis_name="s")
    cp = pltpu.CompilerParams()
    if "needs_layout_passes" in pltpu.CompilerParams.__dataclass_fields__:
        cp = dataclasses.replace(cp, needs_layout_passes=False)
    if "use_tc_tiling_on_sc" in pltpu.CompilerParams.__dataclass_fields__:
        cp = dataclasses.replace(cp, use_tc_tiling_on_sc=False)

    @functools.partial(
        pl.kernel,
        compiler_params=cp,
        out_type=jax.ShapeDtypeStruct((NCORE, NPAD, HW), jnp.float32),
        mesh=mesh,
        scratch_types=[
            pltpu.VMEM_SHARED((NPAD, HW), jnp.float32),
            pltpu.VMEM_SHARED((NDEN, 16), jnp.float32),
        ] + [pltpu.VMEM((WIN,), jnp.int32)] * 8
          + [pltpu.VMEM((WIN, HW), jnp.float32)] * 6
          + [pltpu.VMEM((WIN, 16), jnp.float32)] * 2
          + [pltpu.VMEM((HW, Dh), jnp.float32)]
          + [pltpu.SemaphoreType.DMA] * 6,
    )
    def body(zl_r, zr_r, srcs_r, dsts_r, attp_r, zn_r, zd_r, out_r,
             acc, den, si0, si1, di0, di1, dc0, dc1, d80, d81,
             xi0, xi1, xj0, xj1, co0, co1, db0, db1, attv,
             is0, is1, gs0, gs1, ss0, ss1):
        c = lax.axis_index("c")
        s = lax.axis_index("s")
        iota = lax.iota(jnp.int32, 16)
        sidx = [si0, si1]
        didx = [di0, di1]
        dcopy = [dc0, dc1]
        didx8 = [d80, d81]
        xi = [xi0, xi1]
        xj = [xj0, xj1]
        contrib = [co0, co1]
        dbuf = [db0, db1]
        isem = [is0, is1]
        gsem = [gs0, gs1]
        ssem = [ss0, ss1]

        pltpu.sync_copy(attp_r.at[c], attv)

        @pl.loop(0, RWIN_PER_SUB)
        def _z(it):
            rw = s + it * NSUB

            @pl.when(rw < NRWIN)
            def _():
                rb = pl.multiple_of(rw * RWIN, RWIN)
                pltpu.sync_copy(zn_r.at[pl.ds(rb, RWIN)],
                                acc.at[pl.ds(rb, RWIN)])
                db = pl.multiple_of(rw * (RWIN // 8), RWIN // 8)
                pltpu.sync_copy(zd_r.at[pl.ds(db, RWIN // 8)],
                                den.at[pl.ds(db, RWIN // 8)])

        plsc.subcore_barrier()

        def issue_idx(p, it):
            eb = pl.multiple_of((s + it * NSUB) * WIN, WIN)
            pltpu.async_copy(srcs_r.at[pl.ds(eb, WIN)], sidx[p], isem[p])
            pltpu.async_copy(dsts_r.at[pl.ds(eb, WIN)], didx[p], isem[p])

        def wait_idx(p):
            pltpu.make_async_copy(
                srcs_r.at[pl.ds(0, WIN)], sidx[p], isem[p]).wait()
            pltpu.make_async_copy(
                dsts_r.at[pl.ds(0, WIN)], didx[p], isem[p]).wait()

        def issue_gather(p):
            pltpu.async_copy(zl_r.at[c].at[sidx[p]], xj[p], gsem[p])
            pltpu.async_copy(zr_r.at[c].at[didx[p]], xi[p], gsem[p])

        def wait_gather(p):
            pltpu.make_async_copy(
                zl_r.at[c].at[sidx[p]], xj[p], gsem[p]).wait()
            pltpu.make_async_copy(
                zr_r.at[c].at[didx[p]], xi[p], gsem[p]).wait()

        def issue_scatter(p):
            pltpu.async_copy(contrib[p], acc.at[dcopy[p]], ssem[p], add=True)
            pltpu.async_copy(dbuf[p], den.at[didx8[p]], ssem[p], add=True)

        def wait_scatter(p):
            pltpu.make_async_copy(
                contrib[p], acc.at[dcopy[p]], ssem[p]).wait()
            pltpu.make_async_copy(
                dbuf[p], den.at[didx8[p]], ssem[p]).wait()

        def compute(p):
            @pl.loop(0, WIN // 16)
            def _grp(g):
                rows = g * 16 + iota
                dv = didx[p][pl.ds(g * 16, 16)]
                dcopy[p][pl.ds(g * 16, 16)] = dv
                didx8[p][pl.ds(g * 16, 16)] = dv >> 3
                for j in range(16):
                    dbuf[p][g * 16 + j, :] = jnp.zeros((16,), jnp.float32)
                dlo = (dv & 7) * 2
                for h in range(2):
                    alpha = jnp.zeros((16,), jnp.float32)
                    for k in range(16):
                        col = jnp.full((16,), h * Dh + k, jnp.int32)
                        vi = plsc.load_gather(xi[p], [rows, col])
                        vj = plsc.load_gather(xj[p], [rows, col])
                        v = vi + vj
                        t = jnp.maximum(v, 0.0) + 0.2 * jnp.minimum(v, 0.0)
                        alpha = alpha + attv[h * Dh + k] * t
                    ex = jnp.exp(alpha)
                    plsc.store_scatter(dbuf[p], [rows, dlo + h], ex)
                    for k in range(16):
                        col = jnp.full((16,), h * Dh + k, jnp.int32)
                        vj = plsc.load_gather(xj[p], [rows, col])
                        plsc.store_scatter(contrib[p], [rows, col], vj * ex)

        @pl.loop(0, PIPE_ITERS, step=2)
        def _pipe(base):
            for p in range(2):
                it = base + p

                @pl.when(it >= 2)
                def _c():
                    wait_gather(p)

                    @pl.when(it >= 4)
                    def _ws():
                        wait_scatter(p)

                    compute(p)
                    issue_scatter(p)

                @pl.when((it >= 1) & (it <= WPS))
                def _g():
                    wait_idx(1 - p)
                    issue_gather(1 - p)

                @pl.when(it < WPS)
                def _i():
                    issue_idx(p, it)

        wait_scatter(0)
        wait_scatter(1)
        plsc.subcore_barrier()

        @pl.loop(0, RWIN_PER_SUB)
        def _rd(it):
            rw = s + it * NSUB

            @pl.when(rw < NRWIN)
            def _():
                rb = pl.multiple_of(rw * RWIN, RWIN)
                db = pl.multiple_of(rw * (RWIN // 8), RWIN // 8)
                pltpu.sync_copy(den.at[pl.ds(db, RWIN // 8)],
                                db0.at[pl.ds(0, RWIN // 8)])
                for q in range(4):
                    qb = pl.multiple_of(rb + q * (RWIN // 4), RWIN // 4)
                    pltpu.sync_copy(acc.at[pl.ds(qb, RWIN // 4)], xi0)

                    @pl.loop(0, RWIN // 4)
                    def _row(r):
                        rg = q * (RWIN // 4) + r
                        rsp = jnp.full((16,), 0, jnp.int32) + (rg >> 3)
                        for h in range(2):
                            csp = jnp.full((16,), 0, jnp.int32) + \
                                ((rg & 7) * 2 + h)
                            dsp = plsc.load_gather(db0, [rsp, csp])
                            sl = (r, pl.ds(h * Dh, Dh))
                            xi0[sl] = xi0[sl] / (dsp + 1e-16)

                    pltpu.sync_copy(xi0, out_r.at[c].at[pl.ds(qb, RWIN // 4)])

    return body(zl, zr, srcs, dsts, attp, zn, zd)


def _split_body(z, ref):
    ref[0] = z[:, :HW]
    ref[1] = z[:, HW:]


_W_SPEC = pl.BlockSpec((D, D), lambda i: (0, 0))
_V_SPEC = pl.BlockSpec((1, D), lambda i: (0, 0))
_H_SPEC = pl.BlockSpec((BLK, D), lambda i: (i, 0))
_Z_SPEC = pl.BlockSpec((2, BLK, HW), lambda i: (0, i, 0))
_Z_SHAPE = jax.ShapeDtypeStruct((2, NPAD, HW), jnp.float32)


def _tc_pre(x, Wl0, bl0, Wr0, br0):
    def body(x_ref, wl_ref, bl_ref, wr_ref, br_ref, zl_ref, zr_ref):
        xb = x_ref[...]
        zl = jnp.dot(xb, wl_ref[...], preferred_element_type=jnp.float32) + bl_ref[...]
        zr = jnp.dot(xb, wr_ref[...], preferred_element_type=jnp.float32) + br_ref[...]
        _split_body(zl, zl_ref)
        _split_body(zr, zr_ref)

    return pl.pallas_call(
        body, grid=(GRID,),
        in_specs=[_H_SPEC, _W_SPEC, _V_SPEC, _W_SPEC, _V_SPEC],
        out_specs=[_Z_SPEC, _Z_SPEC],
        out_shape=[_Z_SHAPE, _Z_SHAPE],
    )(x, Wl0, bl0.reshape(1, D), Wr0, br0.reshape(1, D))


def _norm_block(h_ref, sc_ref, bias_ref, gamma_ref, beta_ref):
    agg = jnp.concatenate([sc_ref[0], sc_ref[1]], axis=-1)
    out = agg + bias_ref[...] + h_ref[...]
    mu = jnp.mean(out, axis=-1, keepdims=True)
    var = jnp.mean((out - mu) ** 2, axis=-1, keepdims=True)
    return (out - mu) * lax.rsqrt(var + 1e-5) * gamma_ref[...] + beta_ref[...]


def _tc_mid(h, scout, bias_i, gamma_i, beta_i, Wln, bln, Wrn, brn):
    def body(h_ref, sc_ref, bias_ref, gamma_ref, beta_ref,
             wl_ref, bl_ref, wr_ref, br_ref, hn_ref, zl_ref, zr_ref):
        hn = _norm_block(h_ref, sc_ref, bias_ref, gamma_ref, beta_ref)
        hn_ref[...] = hn
        zl = jnp.dot(hn, wl_ref[...], preferred_element_type=jnp.float32) + bl_ref[...]
        zr = jnp.dot(hn, wr_ref[...], preferred_element_type=jnp.float32) + br_ref[...]
        _split_body(zl, zl_ref)
        _split_body(zr, zr_ref)

    return pl.pallas_call(
        body, grid=(GRID,),
        in_specs=[_H_SPEC, _Z_SPEC, _V_SPEC, _V_SPEC, _V_SPEC,
                  _W_SPEC, _V_SPEC, _W_SPEC, _V_SPEC],
        out_specs=[_H_SPEC, _Z_SPEC, _Z_SPEC],
        out_shape=[jax.ShapeDtypeStruct((N, D), jnp.float32), _Z_SHAPE, _Z_SHAPE],
    )(h, scout, bias_i.reshape(1, D), gamma_i.reshape(1, D), beta_i.reshape(1, D),
      Wln, bln.reshape(1, D), Wrn, brn.reshape(1, D))


def _tc_post(h, scout, bias_i, gamma_i, beta_i, Wo, bo):
    def body(h_ref, sc_ref, bias_ref, gamma_ref, beta_ref,
             wo_ref, bo_ref, out_ref, acc_ref):
        hn = _norm_block(h_ref, sc_ref, bias_ref, gamma_ref, beta_ref)
        psum = jnp.sum(hn, axis=0, keepdims=True)
        i = pl.program_id(0)

        @pl.when(i == 0)
        def _():
            acc_ref[...] = psum

        @pl.when(i > 0)
        def _():
            acc_ref[...] += psum

        @pl.when(i == GRID - 1)
        def _():
            pooled = acc_ref[...] * (1.0 / N)
            out_ref[...] = (jnp.dot(pooled, wo_ref[...],
                                    preferred_element_type=jnp.float32)
                            + bo_ref[...])

    return pl.pallas_call(
        body, grid=(GRID,),
        in_specs=[_H_SPEC, _Z_SPEC, _V_SPEC, _V_SPEC, _V_SPEC, _W_SPEC, _V_SPEC],
        out_specs=pl.BlockSpec((1, D), lambda i: (0, 0)),
        out_shape=jax.ShapeDtypeStruct((1, D), jnp.float32),
        scratch_shapes=[pltpu.VMEM((1, D), jnp.float32)],
    )(h, scout, bias_i.reshape(1, D), gamma_i.reshape(1, D),
      beta_i.reshape(1, D), Wo, bo.reshape(1, D))


def kernel(x, edge_index, Wl, bl, Wr, br, att, bias, gamma, beta, Wo, bo):
    srcs = jnp.concatenate(
        [edge_index[0], jnp.zeros((EPAD - E,), jnp.int32)])
    dsts = jnp.concatenate(
        [edge_index[1], jnp.full((EPAD - E,), NPAD - 1, jnp.int32)])
    zn = jnp.zeros((NPAD, HW), jnp.float32)
    zd = jnp.zeros((NDEN, 16), jnp.float32)

    h = x
    zl_t, zr_t = _tc_pre(x, Wl[0], bl[0], Wr[0], br[0])
    out = None
    for i in range(L):
        a2 = att[i].reshape(NCORE, 2, Dh)
        attp = jnp.broadcast_to(a2[..., None], (NCORE, 2, Dh, 16))
        attp = attp.reshape(NCORE, HW, Dh)
        scout = _sc_edge_pass(zl_t, zr_t, srcs, dsts, attp, zn, zd)
        if i < L - 1:
            h, zl_t, zr_t = _tc_mid(h, scout, bias[i], gamma[i], beta[i],
                                    Wl[i + 1], bl[i + 1], Wr[i + 1], br[i + 1])
        else:
            out = _tc_post(h, scout, bias[i], gamma[i], beta[i], Wo, bo)
    return out


# speedup vs baseline: 27.5235x; 1.2420x over previous
import dataclasses
import functools

import jax
import jax.numpy as jnp
from jax import lax
from jax.experimental import pallas as pl
from jax.experimental.pallas import tpu as pltpu
from jax.experimental.pallas import tpu_sc as plsc

N = 50000
E = 800000
D = 64
H = 4
Dh = 16
L = 3

HW = D // 2
NSUB = 16
NCORE = 2
WIN = 64
NPAD = 51200
EPAD = 800768
NWINP = EPAD // WIN
WPS = NWINP // NSUB
PIPE_ITERS = WPS + 2
RWIN = 256
NRWIN = NPAD // RWIN
RWIN_PER_SUB = -(-NRWIN // NSUB)
NDEN = NPAD // 8

BLK = 2000
GRID = N // BLK


def _sc_edge_pass(zl, zr, srcs, dsts, attp, zn, zd):
    mesh = plsc.VectorSubcoreMesh(core_axis_name="c", subcore_axis_name="s")
    cp = pltpu.CompilerParams()
    if "needs_layout_passes" in pltpu.CompilerParams.__dataclass_fields__:
        cp = dataclasses.replace(cp, needs_layout_passes=False)
    if "use_tc_tiling_on_sc" in pltpu.CompilerParams.__dataclass_fields__:
        cp = dataclasses.replace(cp, use_tc_tiling_on_sc=False)

    @functools.partial(
        pl.kernel,
        compiler_params=cp,
        out_type=jax.ShapeDtypeStruct((NCORE, NPAD, HW), jnp.float32),
        mesh=mesh,
        scratch_types=[
            pltpu.VMEM_SHARED((NPAD, HW), jnp.float32),
            pltpu.VMEM_SHARED((NDEN, 16), jnp.float32),
        ] + [pltpu.VMEM((WIN,), jnp.int32)] * 8
          + [pltpu.VMEM((WIN, HW), jnp.float32)] * 6
          + [pltpu.VMEM((WIN, 16), jnp.float32)] * 2
          + [pltpu.VMEM((2, Dh), jnp.float32)]
          + [pltpu.SemaphoreType.DMA] * 6,
    )
    def body(zl_r, zr_r, srcs_r, dsts_r, attp_r, zn_r, zd_r, out_r,
             acc, den, si0, si1, di0, di1, dc0, dc1, d80, d81,
             xi0, xi1, xj0, xj1, co0, co1, db0, db1, attv,
             is0, is1, gs0, gs1, ss0, ss1):
        c = lax.axis_index("c")
        s = lax.axis_index("s")
        iota = lax.iota(jnp.int32, 16)
        sidx = [si0, si1]
        didx = [di0, di1]
        dcopy = [dc0, dc1]
        didx8 = [d80, d81]
        xi = [xi0, xi1]
        xj = [xj0, xj1]
        contrib = [co0, co1]
        dbuf = [db0, db1]
        isem = [is0, is1]
        gsem = [gs0, gs1]
        ssem = [ss0, ss1]

        pltpu.sync_copy(attp_r.at[c], attv)

        @pl.loop(0, RWIN_PER_SUB)
        def _z(it):
            rw = s + it * NSUB

            @pl.when(rw < NRWIN)
            def _():
                rb = pl.multiple_of(rw * RWIN, RWIN)
                pltpu.sync_copy(zn_r.at[pl.ds(rb, RWIN)],
                                acc.at[pl.ds(rb, RWIN)])
                db = pl.multiple_of(rw * (RWIN // 8), RWIN // 8)
                pltpu.sync_copy(zd_r.at[pl.ds(db, RWIN // 8)],
                                den.at[pl.ds(db, RWIN // 8)])

        plsc.subcore_barrier()

        def issue_idx(p, it):
            eb = pl.multiple_of((s + it * NSUB) * WIN, WIN)
            pltpu.async_copy(srcs_r.at[pl.ds(eb, WIN)], sidx[p], isem[p])
            pltpu.async_copy(dsts_r.at[pl.ds(eb, WIN)], didx[p], isem[p])

        def wait_idx(p):
            pltpu.make_async_copy(
                srcs_r.at[pl.ds(0, WIN)], sidx[p], isem[p]).wait()
            pltpu.make_async_copy(
                dsts_r.at[pl.ds(0, WIN)], didx[p], isem[p]).wait()

        def issue_gather(p):
            pltpu.async_copy(zl_r.at[c].at[sidx[p]], xj[p], gsem[p])
            pltpu.async_copy(zr_r.at[c].at[didx[p]], xi[p], gsem[p])

        def wait_gather(p):
            pltpu.make_async_copy(
                zl_r.at[c].at[sidx[p]], xj[p], gsem[p]).wait()
            pltpu.make_async_copy(
                zr_r.at[c].at[didx[p]], xi[p], gsem[p]).wait()

        def issue_scatter(p):
            pltpu.async_copy(contrib[p], acc.at[dcopy[p]], ssem[p], add=True)
            pltpu.async_copy(dbuf[p], den.at[didx8[p]], ssem[p], add=True)

        def wait_scatter(p):
            pltpu.make_async_copy(
                contrib[p], acc.at[dcopy[p]], ssem[p]).wait()
            pltpu.make_async_copy(
                dbuf[p], den.at[didx8[p]], ssem[p]).wait()

        lane0 = iota == 0

        def compute(p):
            av = [attv[0, pl.ds(0, Dh)], attv[1, pl.ds(0, Dh)]]

            @pl.loop(0, WIN // 16)
            def _grp(g):
                dv = didx[p][pl.ds(g * 16, 16)]
                dcopy[p][pl.ds(g * 16, 16)] = dv
                didx8[p][pl.ds(g * 16, 16)] = dv >> 3
                dlo = (dv & 7) * 2
                for j in range(16):
                    e = g * 16 + j
                    dbuf[p][e, :] = jnp.zeros((16,), jnp.float32)
                for j in range(16):
                    e = g * 16 + j
                    dcol = dlo[j]
                    for h in range(2):
                        vi = xi[p][e, pl.ds(h * Dh, Dh)]
                        vj = xj[p][e, pl.ds(h * Dh, Dh)]
                        v = vi + vj
                        t = jnp.maximum(v, 0.0) + 0.2 * jnp.minimum(v, 0.0)
                        asum = jnp.sum(av[h] * t)
                        ex = jnp.exp(jnp.zeros((16,), jnp.float32) + asum)
                        contrib[p][e, pl.ds(h * Dh, Dh)] = vj * ex
                        plsc.store_scatter(
                            dbuf[p],
                            [jnp.full((16,), 0, jnp.int32) + e,
                             jnp.full((16,), 0, jnp.int32) + (dcol + h)],
                            ex, mask=lane0)

        @pl.loop(0, PIPE_ITERS, step=2)
        def _pipe(base):
            for p in range(2):
                it = base + p

                @pl.when(it >= 2)
                def _c():
                    wait_gather(p)

                    @pl.when(it >= 4)
                    def _ws():
                        wait_scatter(p)

                    compute(p)
                    issue_scatter(p)

                @pl.when((it >= 1) & (it <= WPS))
                def _g():
                    wait_idx(1 - p)
                    issue_gather(1 - p)

                @pl.when(it < WPS)
                def _i():
                    issue_idx(p, it)

        wait_scatter(0)
        wait_scatter(1)
        plsc.subcore_barrier()

        @pl.loop(0, RWIN_PER_SUB)
        def _rd(it):
            rw = s + it * NSUB

            @pl.when(rw < NRWIN)
            def _():
                rb = pl.multiple_of(rw * RWIN, RWIN)
                db = pl.multiple_of(rw * (RWIN // 8), RWIN // 8)
                pltpu.sync_copy(den.at[pl.ds(db, RWIN // 8)],
                                db0.at[pl.ds(0, RWIN // 8)])
                for q in range(4):
                    qb = pl.multiple_of(rb + q * (RWIN // 4), RWIN // 4)
                    pltpu.sync_copy(acc.at[pl.ds(qb, RWIN // 4)], xi0)

                    @pl.loop(0, RWIN // 4)
                    def _row(r):
                        rg = q * (RWIN // 4) + r
                        rsp = jnp.full((16,), 0, jnp.int32) + (rg >> 3)
                        for h in range(2):
                            csp = jnp.full((16,), 0, jnp.int32) + \
                                ((rg & 7) * 2 + h)
                            dsp = plsc.load_gather(db0, [rsp, csp])
                            sl = (r, pl.ds(h * Dh, Dh))
                            xi0[sl] = xi0[sl] / (dsp + 1e-16)

                    pltpu.sync_copy(xi0, out_r.at[c].at[pl.ds(qb, RWIN // 4)])

    return body(zl, zr, srcs, dsts, attp, zn, zd)


def _split_body(z, ref):
    ref[0] = z[:, :HW]
    ref[1] = z[:, HW:]


_W_SPEC = pl.BlockSpec((D, D), lambda i: (0, 0))
_V_SPEC = pl.BlockSpec((1, D), lambda i: (0, 0))
_H_SPEC = pl.BlockSpec((BLK, D), lambda i: (i, 0))
_Z_SPEC = pl.BlockSpec((2, BLK, HW), lambda i: (0, i, 0))
_Z_SHAPE = jax.ShapeDtypeStruct((2, NPAD, HW), jnp.float32)


def _tc_pre(x, Wl0, bl0, Wr0, br0):
    def body(x_ref, wl_ref, bl_ref, wr_ref, br_ref, zl_ref, zr_ref):
        xb = x_ref[...]
        zl = jnp.dot(xb, wl_ref[...], preferred_element_type=jnp.float32) + bl_ref[...]
        zr = jnp.dot(xb, wr_ref[...], preferred_element_type=jnp.float32) + br_ref[...]
        _split_body(zl, zl_ref)
        _split_body(zr, zr_ref)

    return pl.pallas_call(
        body, grid=(GRID,),
        in_specs=[_H_SPEC, _W_SPEC, _V_SPEC, _W_SPEC, _V_SPEC],
        out_specs=[_Z_SPEC, _Z_SPEC],
        out_shape=[_Z_SHAPE, _Z_SHAPE],
    )(x, Wl0, bl0.reshape(1, D), Wr0, br0.reshape(1, D))


def _norm_block(h_ref, sc_ref, bias_ref, gamma_ref, beta_ref):
    agg = jnp.concatenate([sc_ref[0], sc_ref[1]], axis=-1)
    out = agg + bias_ref[...] + h_ref[...]
    mu = jnp.mean(out, axis=-1, keepdims=True)
    var = jnp.mean((out - mu) ** 2, axis=-1, keepdims=True)
    return (out - mu) * lax.rsqrt(var + 1e-5) * gamma_ref[...] + beta_ref[...]


def _tc_mid(h, scout, bias_i, gamma_i, beta_i, Wln, bln, Wrn, brn):
    def body(h_ref, sc_ref, bias_ref, gamma_ref, beta_ref,
             wl_ref, bl_ref, wr_ref, br_ref, hn_ref, zl_ref, zr_ref):
        hn = _norm_block(h_ref, sc_ref, bias_ref, gamma_ref, beta_ref)
        hn_ref[...] = hn
        zl = jnp.dot(hn, wl_ref[...], preferred_element_type=jnp.float32) + bl_ref[...]
        zr = jnp.dot(hn, wr_ref[...], preferred_element_type=jnp.float32) + br_ref[...]
        _split_body(zl, zl_ref)
        _split_body(zr, zr_ref)

    return pl.pallas_call(
        body, grid=(GRID,),
        in_specs=[_H_SPEC, _Z_SPEC, _V_SPEC, _V_SPEC, _V_SPEC,
                  _W_SPEC, _V_SPEC, _W_SPEC, _V_SPEC],
        out_specs=[_H_SPEC, _Z_SPEC, _Z_SPEC],
        out_shape=[jax.ShapeDtypeStruct((N, D), jnp.float32), _Z_SHAPE, _Z_SHAPE],
    )(h, scout, bias_i.reshape(1, D), gamma_i.reshape(1, D), beta_i.reshape(1, D),
      Wln, bln.reshape(1, D), Wrn, brn.reshape(1, D))


def _tc_post(h, scout, bias_i, gamma_i, beta_i, Wo, bo):
    def body(h_ref, sc_ref, bias_ref, gamma_ref, beta_ref,
             wo_ref, bo_ref, out_ref, acc_ref):
        hn = _norm_block(h_ref, sc_ref, bias_ref, gamma_ref, beta_ref)
        psum = jnp.sum(hn, axis=0, keepdims=True)
        i = pl.program_id(0)

        @pl.when(i == 0)
        def _():
            acc_ref[...] = psum

        @pl.when(i > 0)
        def _():
            acc_ref[...] += psum

        @pl.when(i == GRID - 1)
        def _():
            pooled = acc_ref[...] * (1.0 / N)
            out_ref[...] = (jnp.dot(pooled, wo_ref[...],
                                    preferred_element_type=jnp.float32)
                            + bo_ref[...])

    return pl.pallas_call(
        body, grid=(GRID,),
        in_specs=[_H_SPEC, _Z_SPEC, _V_SPEC, _V_SPEC, _V_SPEC, _W_SPEC, _V_SPEC],
        out_specs=pl.BlockSpec((1, D), lambda i: (0, 0)),
        out_shape=jax.ShapeDtypeStruct((1, D), jnp.float32),
        scratch_shapes=[pltpu.VMEM((1, D), jnp.float32)],
    )(h, scout, bias_i.reshape(1, D), gamma_i.reshape(1, D),
      beta_i.reshape(1, D), Wo, bo.reshape(1, D))


def kernel(x, edge_index, Wl, bl, Wr, br, att, bias, gamma, beta, Wo, bo):
    srcs = jnp.concatenate(
        [edge_index[0], jnp.zeros((EPAD - E,), jnp.int32)])
    dsts = jnp.concatenate(
        [edge_index[1], jnp.full((EPAD - E,), NPAD - 1, jnp.int32)])
    zn = jnp.zeros((NPAD, HW), jnp.float32)
    zd = jnp.zeros((NDEN, 16), jnp.float32)

    h = x
    zl_t, zr_t = _tc_pre(x, Wl[0], bl[0], Wr[0], br[0])
    out = None
    for i in range(L):
        attp = att[i].reshape(NCORE, 2, Dh)
        scout = _sc_edge_pass(zl_t, zr_t, srcs, dsts, attp, zn, zd)
        if i < L - 1:
            h, zl_t, zr_t = _tc_mid(h, scout, bias[i], gamma[i], beta[i],
                                    Wl[i + 1], bl[i + 1], Wr[i + 1], br[i + 1])
        else:
            out = _tc_post(h, scout, bias[i], gamma[i], beta[i], Wo, bo)
    return out
